# baseline (speedup 1.0000x reference)
import jax
import jax.numpy as jnp
from jax import lax
from jax.experimental import pallas as pl
from jax.experimental.pallas import tpu as pltpu
from jax.experimental.pallas import tpu_sc as plsc

_NC = 2
_NS = 16
_NW = _NC * _NS

_K = 128
_IC = 12800
_ISTEPS = _IC // _K
_NB = 4
_AHEAD = 2


def kernel(poi_categories, table):
    batch, seq = poi_categories.shape
    _, dim = table.shape
    n = batch * seq
    per_w = n // _NW
    nsc = per_w // _IC
    idx = poi_categories.reshape(n).astype(jnp.int32)

    mesh = plsc.VectorSubcoreMesh(core_axis_name="c", subcore_axis_name="s")

    @pl.kernel(
        out_type=jax.ShapeDtypeStruct((n, dim), table.dtype),
        mesh=mesh,
        scratch_types=[
            pltpu.VMEM((2, _IC), jnp.int32),
            pltpu.VMEM((_NB, _K, dim), jnp.float32),
            pltpu.SemaphoreType.DMA((2,)),
            pltpu.SemaphoreType.DMA((_NB,)),
            pltpu.SemaphoreType.DMA((_NB,)),
        ],
    )
    def _gather(table_hbm, idx_hbm, out_hbm, idx_v, rows_v, isem, gsem, wsem):
        wid = lax.axis_index("s") * _NC + lax.axis_index("c")
        base = wid * per_w

        def idx_load(c, jc):
            return pltpu.make_async_copy(
                idx_hbm.at[pl.ds(base + c * _IC, _IC)], idx_v.at[jc], isem.at[jc]
            )

        def gather(jc, s, slot):
            src = table_hbm.at[idx_v.at[jc, pl.ds(s * _K, _K)]]
            return pltpu.make_async_copy(src, rows_v.at[slot], gsem.at[slot])

        def writeback(cb, s, slot):
            dst = out_hbm.at[pl.ds(cb + s * _K, _K)]
            return pltpu.make_async_copy(rows_v.at[slot], dst, wsem.at[slot])

        idx_load(0, 0).start()

        @pl.loop(0, nsc)
        def _(c):
            jc = lax.rem(c, 2)
            idx_load(c, jc).wait()

            @pl.when(c + 1 < nsc)
            def _():
                idx_load(c + 1, lax.rem(c + 1, 2)).start()

            chunk_base = base + c * _IC

            for b in range(_AHEAD):
                @pl.when(c > 0)
                def _(b=b):
                    writeback(chunk_base, 0, b).wait()

                gather(jc, b, b).start()

            @pl.loop(0, _ISTEPS, step=_NB)
            def _(r):
                for b in range(_NB):
                    s = r + b
                    slot = b
                    gather(jc, s, slot).wait()
                    writeback(chunk_base, s, slot).start()

                    s2 = s + _AHEAD
                    slot2 = (b + _AHEAD) % _NB
                    if b < _NB - _AHEAD:
                        @pl.when((c > 0) | (s2 >= _NB))
                        def _(slot2=slot2):
                            writeback(chunk_base, 0, slot2).wait()

                        gather(jc, s2, slot2).start()
                    else:
                        @pl.when(s2 < _ISTEPS)
                        def _(s2=s2, slot2=slot2):
                            writeback(chunk_base, 0, slot2).wait()
                            gather(jc, s2, slot2).start()

        for b in range(_NB):
            writeback(0, 0, b).wait()

    out = _gather(table, idx)
    return out.reshape(batch, seq, dim)


# speedup vs baseline: 1.4017x; 1.4017x over previous
import jax
import jax.numpy as jnp
from jax.experimental import pallas as pl
from jax.experimental.pallas import tpu as pltpu
from jax.experimental.pallas import tpu_sc as plsc

_WINDOW = 128


def kernel(poi_categories, table):
    batch, seq = poi_categories.shape
    vocab, dim = table.shape
    n = batch * seq
    np_ = n // 2

    idx = poi_categories.reshape(np_, 2).astype(jnp.int32)
    pid = (idx[:, 0] * vocab + idx[:, 1]).reshape(1, np_)
    table2 = jnp.concatenate(
        [
            jnp.broadcast_to(table[:, None, :], (vocab, vocab, dim)),
            jnp.broadcast_to(table[None, :, :], (vocab, vocab, dim)),
        ],
        axis=-1,
    ).reshape(vocab * vocab, 2 * dim)

    mesh = plsc.VectorSubcoreMesh(core_axis_name="c", subcore_axis_name="s")

    @pl.kernel(out_type=jax.ShapeDtypeStruct((np_, 2 * dim), table.dtype), mesh=mesh)
    def _gather(table_hbm, idx_hbm, out_hbm):
        def body(i_vmem, o_vmem):
            pltpu.sync_copy(table_hbm.at[i_vmem.at[0]], o_vmem)

        pltpu.emit_pipeline(
            body,
            grid=(np_ // _WINDOW,),
            in_specs=[pl.BlockSpec((1, _WINDOW), index_map=lambda i: (0, i))],
            out_specs=[pl.BlockSpec((_WINDOW, 2 * dim), index_map=lambda i: (i, 0))],
            core_axis_name=("c", "s"),
            dimension_semantics=(pltpu.PARALLEL,),
        )(idx_hbm, out_hbm)

    out = _gather(table2, pid)
    return out.reshape(batch, seq, dim)


# speedup vs baseline: 3.8900x; 2.7753x over previous
import jax
import jax.numpy as jnp
from jax import lax
from jax.experimental import pallas as pl
from jax.experimental.pallas import tpu as pltpu
from jax.experimental.pallas import tpu_sc as plsc

_WINDOW = 128
_R = 2048
_TC_FRAC = 1.0


def _tc_lookup(idx_tc, table, n_out):
    m = idx_tc.shape[0]
    vocab, dim = table.shape
    nblk = m // _R
    idx3 = idx_tc.reshape(nblk, 1, _R)

    tpad = jnp.zeros((128, dim), table.dtype).at[:vocab].set(table)
    thi = tpad.astype(jnp.bfloat16)
    tlo = (tpad - thi.astype(jnp.float32)).astype(jnp.bfloat16)

    def body(idx_ref, thi_ref, tlo_ref, o_ref):
        ids = idx_ref[0, 0, :]
        oh = (ids[:, None] == lax.broadcasted_iota(jnp.int32, (_R, 128), 1)).astype(
            jnp.bfloat16
        )
        o_ref[...] = jnp.dot(
            oh, thi_ref[...], preferred_element_type=jnp.float32
        ) + jnp.dot(oh, tlo_ref[...], preferred_element_type=jnp.float32)

    return pl.pallas_call(
        body,
        grid=(nblk,),
        in_specs=[
            pl.BlockSpec((1, 1, _R), lambda i: (i, 0, 0)),
            pl.BlockSpec((128, dim), lambda i: (0, 0)),
            pl.BlockSpec((128, dim), lambda i: (0, 0)),
        ],
        out_specs=pl.BlockSpec((_R, dim), lambda i: (i, 0)),
        out_shape=jax.ShapeDtypeStruct((n_out, dim), table.dtype),
    )(idx3, thi, tlo)


def _sc_lookup(idx_sc, table):
    m = idx_sc.shape[0]
    vocab, dim = table.shape
    np_ = m // 2

    pid = (idx_sc.reshape(np_, 2)[:, 0] * vocab + idx_sc.reshape(np_, 2)[:, 1]).reshape(
        1, np_
    )
    table2 = jnp.concatenate(
        [
            jnp.broadcast_to(table[:, None, :], (vocab, vocab, dim)),
            jnp.broadcast_to(table[None, :, :], (vocab, vocab, dim)),
        ],
        axis=-1,
    ).reshape(vocab * vocab, 2 * dim)

    mesh = plsc.VectorSubcoreMesh(core_axis_name="c", subcore_axis_name="s")

    @pl.kernel(out_type=jax.ShapeDtypeStruct((np_, 2 * dim), table.dtype), mesh=mesh)
    def _gather(table_hbm, idx_hbm, out_hbm):
        def body(i_vmem, o_vmem):
            pltpu.sync_copy(table_hbm.at[i_vmem.at[0]], o_vmem)

        pltpu.emit_pipeline(
            body,
            grid=(np_ // _WINDOW,),
            in_specs=[pl.BlockSpec((1, _WINDOW), index_map=lambda i: (0, i))],
            out_specs=[pl.BlockSpec((_WINDOW, 2 * dim), index_map=lambda i: (i, 0))],
            core_axis_name=("c", "s"),
            dimension_semantics=(pltpu.PARALLEL,),
        )(idx_hbm, out_hbm)

    return _gather(table2, pid).reshape(m, dim)


def kernel(poi_categories, table):
    batch, seq = poi_categories.shape
    vocab, dim = table.shape
    n = batch * seq
    idx = poi_categories.reshape(n).astype(jnp.int32)

    n_tc = int(n * _TC_FRAC) // 8192 * 8192
    n_sc = n - n_tc

    if n_sc == 0:
        out = _tc_lookup(idx, table, n)
    elif n_tc == 0:
        out = _sc_lookup(idx, table)
    else:
        out_tc = _tc_lookup(idx[:n_tc], table, n)
        out_sc = _sc_lookup(idx[n_tc:], table)
        out = lax.dynamic_update_slice(out_tc, out_sc, (n_tc, 0))
    return out.reshape(batch, seq, dim)
